# baseline (speedup 1.0000x reference)
import jax
import jax.numpy as jnp
from jax import lax
from jax.experimental import pallas as pl
from jax.experimental.pallas import tpu as pltpu
from jax.experimental.pallas import tpu_sc as plsc

VOCAB = 1000000
EMBDIM = 64
B = 4096
S = 200

NUM_CORES = 2
NUM_SUBCORES = 16
NW = NUM_CORES * NUM_SUBCORES
CPW = B // NW
NB = 4
GROUPS = S // NB


def _body(xt_hbm, table_hbm, out_hbm, idx_v,
          r0, r1, r2, r3, g0, g1, g2, g3, s0, s1, s2, s3):
    rows = (r0, r1, r2, r3)
    gs = (g0, g1, g2, g3)
    ss = (s0, s1, s2, s3)
    wid = lax.axis_index("s") * NUM_CORES + lax.axis_index("c")
    c0 = wid * CPW

    pltpu.sync_copy(xt_hbm.at[:, pl.ds(c0, CPW)], idx_v)

    def fire_g(j, b):
        pltpu.async_copy(table_hbm.at[idx_v.at[j]], rows[b], gs[b])

    def wait_g(j, b):
        pltpu.make_async_copy(table_hbm.at[idx_v.at[j]], rows[b], gs[b]).wait()

    def fire_s(j, b):
        pltpu.async_copy(rows[b], out_hbm.at[pl.ds(c0, CPW), j], ss[b])

    def wait_s(j, b):
        pltpu.make_async_copy(
            rows[b], out_hbm.at[pl.ds(c0, CPW), j], ss[b]).wait()

    def slot(j, b, do_wait_s, do_fire_g):
        wait_g(j, b)
        fire_s(j, b)
        if do_fire_g:
            b2 = (b + 2) % NB
            if do_wait_s:
                wait_s(j - 2, b2)
            fire_g(j + 2, b2)

    fire_g(0, 0)
    fire_g(1, 1)
    slot(0, 0, False, True)
    slot(1, 1, False, True)
    slot(2, 2, True, True)
    slot(3, 3, True, True)

    def group(g, carry):
        j0 = g * NB
        for k in range(NB):
            slot(j0 + k, k, True, True)
        return carry

    lax.fori_loop(1, GROUPS - 1, group, 0)

    j0 = S - NB
    slot(j0 + 0, 0, True, True)
    slot(j0 + 1, 1, True, True)
    slot(j0 + 2, 2, False, False)
    slot(j0 + 3, 3, False, False)
    for k in range(NB):
        wait_s(j0 + k, k)


def kernel(x, table):
    mesh = plsc.VectorSubcoreMesh(core_axis_name="c", subcore_axis_name="s")
    return pl.kernel(
        _body,
        out_type=jax.ShapeDtypeStruct((B, S, EMBDIM), jnp.float32),
        mesh=mesh,
        scratch_types=[
            pltpu.VMEM((S, CPW), jnp.int32),
            pltpu.VMEM((CPW, EMBDIM), jnp.float32),
            pltpu.VMEM((CPW, EMBDIM), jnp.float32),
            pltpu.VMEM((CPW, EMBDIM), jnp.float32),
            pltpu.VMEM((CPW, EMBDIM), jnp.float32),
            pltpu.SemaphoreType.DMA,
            pltpu.SemaphoreType.DMA,
            pltpu.SemaphoreType.DMA,
            pltpu.SemaphoreType.DMA,
            pltpu.SemaphoreType.DMA,
            pltpu.SemaphoreType.DMA,
            pltpu.SemaphoreType.DMA,
            pltpu.SemaphoreType.DMA,
        ],
        compiler_params=pltpu.CompilerParams(use_tc_tiling_on_sc=False),
    )(x.T.astype(jnp.int32), table)


# speedup vs baseline: 1.3402x; 1.3402x over previous
import jax
import jax.numpy as jnp
from jax import lax
from jax.experimental import pallas as pl
from jax.experimental.pallas import tpu as pltpu
from jax.experimental.pallas import tpu_sc as plsc

VOCAB = 1000000
EMBDIM = 64
B = 4096
S = 200

NUM_CORES = 2
NUM_SUBCORES = 16
NW = NUM_CORES * NUM_SUBCORES
CPW = B // NW
NB = 4
GROUPS = S // NB


def _body(xt_hbm, table_hbm, out_hbm, idx_v,
          r0, r1, r2, r3, g0, g1, g2, g3, s0, s1, s2, s3):
    rows = (r0, r1, r2, r3)
    gs = (g0, g1, g2, g3)
    ss = (s0, s1, s2, s3)
    wid = lax.axis_index("s") * NUM_CORES + lax.axis_index("c")
    c0 = wid * CPW

    pltpu.sync_copy(xt_hbm.at[:, pl.ds(c0, CPW)], idx_v)

    def fire_g(j, b):
        pltpu.async_copy(table_hbm.at[idx_v.at[j]], rows[b], gs[b])

    def wait_g(j, b):
        pltpu.make_async_copy(table_hbm.at[idx_v.at[j]], rows[b], gs[b]).wait()

    def fire_s(j, b):
        pltpu.async_copy(rows[b], out_hbm.at[pl.ds(c0, CPW), j], ss[b])

    def wait_s(j, b):
        pltpu.make_async_copy(
            rows[b], out_hbm.at[pl.ds(c0, CPW), j], ss[b]).wait()

    def slot(j, b, do_wait_s, do_fire_g):
        wait_g(j, b)
        fire_s(j, b)
        if do_fire_g:
            b2 = (b + 2) % NB
            if do_wait_s:
                wait_s(j - 2, b2)
            fire_g(j + 2, b2)

    fire_g(0, 0)
    fire_g(1, 1)
    slot(0, 0, False, True)
    slot(1, 1, False, True)
    slot(2, 2, True, True)
    slot(3, 3, True, True)

    def group(g, carry):
        j0 = g * NB
        for k in range(NB):
            slot(j0 + k, k, True, True)
        return carry

    lax.fori_loop(1, GROUPS - 1, group, 0)

    j0 = S - NB
    slot(j0 + 0, 0, True, True)
    slot(j0 + 1, 1, True, True)
    slot(j0 + 2, 2, False, False)
    slot(j0 + 3, 3, False, False)
    for k in range(NB):
        wait_s(j0 + k, k)


def kernel(x, table):
    mesh = plsc.VectorSubcoreMesh(core_axis_name="c", subcore_axis_name="s")
    tpad = jnp.pad(table, ((0, 0), (0, 128 - EMBDIM)))
    out = pl.kernel(
        _body,
        out_type=jax.ShapeDtypeStruct((B, S, 128), jnp.float32),
        mesh=mesh,
        scratch_types=[
            pltpu.VMEM((S, CPW), jnp.int32),
            pltpu.VMEM((CPW, 128), jnp.float32),
            pltpu.VMEM((CPW, 128), jnp.float32),
            pltpu.VMEM((CPW, 128), jnp.float32),
            pltpu.VMEM((CPW, 128), jnp.float32),
            pltpu.SemaphoreType.DMA,
            pltpu.SemaphoreType.DMA,
            pltpu.SemaphoreType.DMA,
            pltpu.SemaphoreType.DMA,
            pltpu.SemaphoreType.DMA,
            pltpu.SemaphoreType.DMA,
            pltpu.SemaphoreType.DMA,
            pltpu.SemaphoreType.DMA,
        ],
        compiler_params=pltpu.CompilerParams(use_tc_tiling_on_sc=True),
    )(x.T.astype(jnp.int32), tpad)
    return out[:, :, :EMBDIM]
